# baseline (speedup 1.0000x reference)
import functools
import jax
import jax.numpy as jnp
from jax import lax
from jax.experimental import pallas as pl
from jax.experimental.pallas import tpu as pltpu
from jax.experimental.pallas import tpu_sc as plsc

B, E_IN, C, D, K, V = 256, 768, 8, 64, 1024, 256
NW = 32
ROWS_PER_W = (B * C) // NW


def _tc_body(batch_ref, rp_ref, w_ref, b_ref, cb_ref, idx_ref):
    c = pl.program_id(0)
    x = batch_ref[...]
    tq = jnp.dot(x, rp_ref[...], preferred_element_type=jnp.float32)
    y = lax.dot_general(tq, w_ref[...], (((1,), (1,)), ((), ())),
                        preferred_element_type=jnp.float32) + b_ref[pl.ds(c, 1), :]
    cb = cb_ref[...]
    xe = lax.dot_general(y, cb, (((1,), (1,)), ((), ())),
                         preferred_element_type=jnp.float32)
    x2 = jnp.sum(y * y, axis=1, keepdims=True)
    e2 = jnp.sum(cb * cb, axis=1)
    dist = -(x2 - 2.0 * xe + e2[None, :])
    m = jnp.max(dist, axis=1, keepdims=True)
    kidx = lax.broadcasted_iota(jnp.int32, (B, K), 1)
    idx = jnp.min(jnp.where(dist == m, kidx, K), axis=1, keepdims=True)
    lane = lax.broadcasted_iota(jnp.int32, (B, C), 1)
    idx_ref[...] = jnp.where(lane == c, idx + c * K, idx_ref[...])


def _sc_gather_mean(gidx_hbm, vflat_hbm, out_hbm, idx_v, rows_v, tmp32, out_v,
                    sem):
    wid = lax.axis_index("s") * 2 + lax.axis_index("c")
    base = wid * ROWS_PER_W
    pltpu.sync_copy(gidx_hbm.at[pl.ds(base, ROWS_PER_W)], idx_v)
    pltpu.async_copy(vflat_hbm.at[idx_v], rows_v, sem).wait()

    lanes = lax.iota(jnp.int32, 16)
    for g in range(ROWS_PER_W // 16):
        grp = jnp.zeros((16,), jnp.float32)
        for l in range(16):
            i = g * 16 + l
            p = rows_v[i, pl.ds(0, 16)]
            for j in range(1, V // 16):
                p = p + rows_v[i, pl.ds(j * 16, 16)]
            for sh in (8, 4, 2, 1):
                tmp32[pl.ds(0, 16)] = p
                tmp32[pl.ds(16, 16)] = p
                p = p + tmp32[pl.ds(sh, 16)]
            grp = jnp.where(lanes == l, p, grp)
        out_v[pl.ds(g * 16, 16)] = grp * (1.0 / V)
    pltpu.sync_copy(out_v, out_hbm.at[pl.ds(base, ROWS_PER_W)])


@jax.jit
def kernel(batch, values, rand_proj, W_in, b_in, codebook):
    gidx = pl.pallas_call(
        _tc_body,
        grid=(C,),
        in_specs=[
            pl.BlockSpec((B, E_IN), lambda c: (0, 0)),
            pl.BlockSpec((E_IN, D), lambda c: (c, 0)),
            pl.BlockSpec((D, D), lambda c: (c, 0)),
            pl.BlockSpec((C, D), lambda c: (0, 0)),
            pl.BlockSpec((K, D), lambda c: (c, 0)),
        ],
        out_specs=pl.BlockSpec((B, C), lambda c: (0, 0)),
        out_shape=jax.ShapeDtypeStruct((B, C), jnp.int32),
    )(batch, rand_proj.reshape(C * E_IN, D), W_in, b_in.reshape(C, D),
      codebook.reshape(C * K, D))

    sc = functools.partial(
        pl.kernel,
        mesh=plsc.VectorSubcoreMesh(core_axis_name="c", subcore_axis_name="s"),
        out_type=jax.ShapeDtypeStruct((B * C,), jnp.float32),
        scratch_types=[
            pltpu.VMEM((ROWS_PER_W,), jnp.int32),
            pltpu.VMEM((ROWS_PER_W, V), jnp.float32),
            pltpu.VMEM((32,), jnp.float32),
            pltpu.VMEM((ROWS_PER_W,), jnp.float32),
            pltpu.SemaphoreType.DMA,
        ],
    )(_sc_gather_mean)
    out_flat = sc(gidx.reshape(B * C), values.reshape(C * K, V))
    return out_flat.reshape(B, C)


# speedup vs baseline: 2.5633x; 2.5633x over previous
import jax
import jax.numpy as jnp
from jax import lax
from jax.experimental import pallas as pl

B, E_IN, C, D, K, V = 256, 768, 8, 64, 1024, 256


def _tc_body(batch_ref, rpT_ref, wT_ref, b_ref, cbT_ref, val_ref, out_ref):
    c = pl.program_id(0)
    x = batch_ref[...]
    rpT = rpT_ref[0]
    tq = lax.dot_general(x, rpT, (((1,), (1,)), ((), ())),
                         preferred_element_type=jnp.float32)
    y = lax.dot_general(tq, wT_ref[...], (((1,), (1,)), ((), ())),
                        preferred_element_type=jnp.float32) \
        + b_ref[pl.ds(c, 1), :]
    cbT = cbT_ref[0]
    xe = jnp.dot(y, cbT, preferred_element_type=jnp.float32)
    x2 = jnp.sum(y * y, axis=1, keepdims=True)
    e2 = jnp.sum(cbT * cbT, axis=0)
    dist = -(x2 - 2.0 * xe + e2[None, :])
    m = jnp.max(dist, axis=1, keepdims=True)
    kidx = lax.broadcasted_iota(jnp.int32, (B, K), 1)
    idx = jnp.min(jnp.where(dist == m, kidx, K), axis=1, keepdims=True)
    ones = jnp.full((V,), 1.0 / V, dtype=jnp.float32)
    vmean = lax.dot_general(val_ref[0], ones, (((1,), (0,)), ((), ())),
                            preferred_element_type=jnp.float32)
    sel = jnp.where(kidx == idx, vmean[None, :], 0.0)
    col = jnp.sum(sel, axis=1, keepdims=True)
    lane = lax.broadcasted_iota(jnp.int32, (B, C), 1)
    out_ref[...] = jnp.where(lane == c, col, out_ref[...])


@jax.jit
def kernel(batch, values, rand_proj, W_in, b_in, codebook):
    out = pl.pallas_call(
        _tc_body,
        grid=(C,),
        in_specs=[
            pl.BlockSpec((B, E_IN), lambda c: (0, 0)),
            pl.BlockSpec((1, D, E_IN), lambda c: (c, 0, 0)),
            pl.BlockSpec((D, D), lambda c: (c, 0)),
            pl.BlockSpec((C, D), lambda c: (0, 0)),
            pl.BlockSpec((1, D, K), lambda c: (c, 0, 0)),
            pl.BlockSpec((1, K, V), lambda c: (c, 0, 0)),
        ],
        out_specs=pl.BlockSpec((B, C), lambda c: (0, 0)),
        out_shape=jax.ShapeDtypeStruct((B, C), jnp.float32),
    )(batch, rand_proj.transpose(0, 2, 1), W_in, b_in.reshape(C, D),
      codebook.transpose(0, 2, 1), values)
    return out
